# Initial kernel scaffold; baseline (speedup 1.0000x reference)
import jax
import jax.numpy as jnp
from jax.experimental import pallas as pl


def kernel(x, edge_index, edge_attr, params):
    raise NotImplementedError("write your pallas kernel here")


# speedup vs baseline: 2.9121x; 2.9121x over previous
import functools

import jax
import jax.numpy as jnp
from jax import lax
from jax.experimental import pallas as pl
from jax.experimental.pallas import tpu as pltpu
from jax.experimental.pallas import tpu_sc as plsc

NC = 2
NS = 16
NW = NC * NS
LANES = 16


def _ea_proj_body(attr_ref, w_ref, b_ref, out_ref):
    out_ref[...] = (
        jnp.dot(attr_ref[...], w_ref[...], preferred_element_type=jnp.float32)
        + b_ref[...]
    )


def _ea_proj(edge_attr, w, b, blk=2000):
    E, ED = edge_attr.shape
    D = w.shape[1]
    return pl.pallas_call(
        _ea_proj_body,
        grid=(E // blk,),
        in_specs=[
            pl.BlockSpec((blk, ED), lambda i: (i, 0)),
            pl.BlockSpec((ED, D), lambda i: (0, 0)),
            pl.BlockSpec((1, D), lambda i: (0, 0)),
        ],
        out_specs=pl.BlockSpec((blk, D), lambda i: (i, 0)),
        out_shape=jax.ShapeDtypeStruct((E, D), jnp.float32),
    )(edge_attr, w, b.reshape(1, D))


def _mlp_body(x_ref, a0_ref, a1_ref, w0, b0, w1, b1, w2, b2, out_ref, *,
              final_relu):
    h = x_ref[...] + a0_ref[...] + a1_ref[...]
    h = jnp.maximum(
        jnp.dot(h, w0[...], preferred_element_type=jnp.float32) + b0[...], 0.0)
    h = jnp.maximum(
        jnp.dot(h, w1[...], preferred_element_type=jnp.float32) + b1[...], 0.0)
    h = jnp.dot(h, w2[...], preferred_element_type=jnp.float32) + b2[...]
    if final_relu:
        h = jnp.maximum(h, 0.0)
    out_ref[...] = h


def _mlp(x, a0, a1, p, final_relu, blk=1000):
    N, D = x.shape
    mat = pl.BlockSpec((D, D), lambda i: (0, 0))
    vec = pl.BlockSpec((1, D), lambda i: (0, 0))
    rows = pl.BlockSpec((blk, D), lambda i: (i, 0))
    return pl.pallas_call(
        functools.partial(_mlp_body, final_relu=final_relu),
        grid=(N // blk,),
        in_specs=[rows, rows, rows, mat, vec, mat, vec, mat, vec],
        out_specs=rows,
        out_shape=jax.ShapeDtypeStruct((N, D), jnp.float32),
    )(x, a0, a1,
      p['W0'], p['b0'].reshape(1, D),
      p['W1'], p['b1'].reshape(1, D),
      p['W2'], p['b2'].reshape(1, D))


@functools.lru_cache(maxsize=None)
def _sc_gather_scatter_fn(N, D, C, NCHG, GG):
    NCH = NCHG * GG
    ZC = C
    NZ = N // ZC
    ZT = (NZ + NS - 1) // NS
    DV = D // LANES

    mesh = plsc.VectorSubcoreMesh(core_axis_name="c", subcore_axis_name="s")

    @functools.partial(
        pl.kernel,
        out_type=[jax.ShapeDtypeStruct((N, D), jnp.float32),
                  jax.ShapeDtypeStruct((N, D), jnp.float32)],
        mesh=mesh,
        scratch_types=[
            pltpu.VMEM((GG, C), jnp.int32),
            pltpu.VMEM((GG, C), jnp.int32),
            pltpu.VMEM((C, D), jnp.float32),
            pltpu.VMEM((C, D), jnp.float32),
            pltpu.VMEM_SHARED((N, D), jnp.float32),
        ],
    )
    def k(x_hbm, srcr_hbm, dstr_hbm, ea_hbm, out0, out1,
          src_v, dst_v, ea_v, g_v, agg_sh):
        c = lax.axis_index("c")
        s = lax.axis_index("s")
        wid = s * NC + c

        zero = jnp.zeros((LANES,), jnp.float32)

        def zrow(r, carry):
            for j in range(DV):
                ea_v[r, pl.ds(j * LANES, LANES)] = zero
            return carry

        lax.fori_loop(0, ZC, zrow, 0)
        for t in range(ZT):
            zc = s + t * NS

            @pl.when(jnp.logical_or(NZ % NS == 0, zc < NZ))
            def _():
                pltpu.sync_copy(ea_v, agg_sh.at[pl.ds(zc * ZC, ZC)])

        plsc.subcore_barrier()

        base = wid * NCH
        for gi in range(NCHG):
            pltpu.sync_copy(srcr_hbm.at[wid, gi], src_v)
            pltpu.sync_copy(dstr_hbm.at[wid, gi], dst_v)

            def chunk(g2, carry):
                g = gi * GG + g2
                pltpu.sync_copy(ea_hbm.at[pl.ds((base + g) * C, C)], ea_v)
                pltpu.sync_copy(x_hbm.at[src_v.at[g2]], g_v)

                def row(r, rc):
                    for j in range(DV):
                        sl = pl.ds(j * LANES, LANES)
                        g_v[r, sl] = jnp.maximum(g_v[r, sl] + ea_v[r, sl], 0.0)
                    return rc

                lax.fori_loop(0, C, row, 0)
                pltpu.sync_copy(g_v, agg_sh.at[dst_v.at[g2]], add=True)
                return carry

            lax.fori_loop(0, GG, chunk, 0)
        plsc.subcore_barrier()

        for t in range(ZT):
            zc = s + t * NS

            @pl.when(jnp.logical_or(NZ % NS == 0, zc < NZ))
            def _():
                row0 = zc * ZC
                pltpu.sync_copy(agg_sh.at[pl.ds(row0, ZC)], ea_v)

                @pl.when(c == 0)
                def _():
                    pltpu.sync_copy(ea_v, out0.at[pl.ds(row0, ZC)])

                @pl.when(c == 1)
                def _():
                    pltpu.sync_copy(ea_v, out1.at[pl.ds(row0, ZC)])

    return k


def kernel(x, edge_index, edge_attr, params):
    N, D = x.shape
    E = edge_index.shape[1]
    C = 80
    NCH = E // (NW * C)
    GG = 25
    NCHG = NCH // GG

    src_r = edge_index[0].reshape(NW, NCHG, GG, C)
    dst_r = edge_index[1].reshape(NW, NCHG, GG, C)

    h = x
    nl = len(params)
    for l, p in enumerate(params):
        ea = _ea_proj(edge_attr, p['eW'], p['eb'])
        sc = _sc_gather_scatter_fn(N, D, C, NCHG, GG)
        a0, a1 = sc(h, src_r, dst_r, ea)
        h = _mlp(h, a0, a1, p, final_relu=(l < nl - 1))
    return h


# speedup vs baseline: 4.5867x; 1.5750x over previous
import functools

import jax
import jax.numpy as jnp
from jax import lax
from jax.experimental import pallas as pl
from jax.experimental.pallas import tpu as pltpu
from jax.experimental.pallas import tpu_sc as plsc

NC = 2
NS = 16
NW = NC * NS
LANES = 16


def _ea_proj_body(attr_ref, w_ref, b_ref, out_ref):
    out_ref[...] = (
        jnp.dot(attr_ref[...], w_ref[...], preferred_element_type=jnp.float32)
        + b_ref[...]
    )


def _ea_proj(edge_attr, w, b, blk=2000):
    E, ED = edge_attr.shape
    D = w.shape[1]
    return pl.pallas_call(
        _ea_proj_body,
        grid=(E // blk,),
        in_specs=[
            pl.BlockSpec((blk, ED), lambda i: (i, 0)),
            pl.BlockSpec((ED, D), lambda i: (0, 0)),
            pl.BlockSpec((1, D), lambda i: (0, 0)),
        ],
        out_specs=pl.BlockSpec((blk, D), lambda i: (i, 0)),
        out_shape=jax.ShapeDtypeStruct((E, D), jnp.float32),
    )(edge_attr, w, b.reshape(1, D))


def _mlp_body(x_ref, a0_ref, a1_ref, w0, b0, w1, b1, w2, b2, out_ref, *,
              final_relu):
    h = x_ref[...] + a0_ref[...] + a1_ref[...]
    h = jnp.maximum(
        jnp.dot(h, w0[...], preferred_element_type=jnp.float32) + b0[...], 0.0)
    h = jnp.maximum(
        jnp.dot(h, w1[...], preferred_element_type=jnp.float32) + b1[...], 0.0)
    h = jnp.dot(h, w2[...], preferred_element_type=jnp.float32) + b2[...]
    if final_relu:
        h = jnp.maximum(h, 0.0)
    out_ref[...] = h


def _mlp(x, a0, a1, p, final_relu, blk=1000):
    N, D = x.shape
    mat = pl.BlockSpec((D, D), lambda i: (0, 0))
    vec = pl.BlockSpec((1, D), lambda i: (0, 0))
    rows = pl.BlockSpec((blk, D), lambda i: (i, 0))
    return pl.pallas_call(
        functools.partial(_mlp_body, final_relu=final_relu),
        grid=(N // blk,),
        in_specs=[rows, rows, rows, mat, vec, mat, vec, mat, vec],
        out_specs=rows,
        out_shape=jax.ShapeDtypeStruct((N, D), jnp.float32),
    )(x, a0, a1,
      p['W0'], p['b0'].reshape(1, D),
      p['W1'], p['b1'].reshape(1, D),
      p['W2'], p['b2'].reshape(1, D))


@functools.lru_cache(maxsize=None)
def _sc_gather_scatter_fn(N, D, C, NCH):
    NB = 3
    NBI = 6
    ZC = C
    NZ = N // ZC
    ZT = (NZ + NS - 1) // NS
    DV = D // LANES
    T = (NCH + NBI - 1) // NBI

    mesh = plsc.VectorSubcoreMesh(core_axis_name="c", subcore_axis_name="s")

    @functools.partial(
        pl.kernel,
        out_type=[jax.ShapeDtypeStruct((N, D), jnp.float32),
                  jax.ShapeDtypeStruct((N, D), jnp.float32)],
        mesh=mesh,
        scratch_types=[
            pltpu.VMEM((NBI, 1, C), jnp.int32),
            pltpu.VMEM((NBI, 1, C), jnp.int32),
            pltpu.VMEM((NB, C, D), jnp.float32),
            pltpu.VMEM((NB, C, D), jnp.float32),
            pltpu.VMEM_SHARED((N, D), jnp.float32),
            [pltpu.SemaphoreType.DMA] * NBI,
            [pltpu.SemaphoreType.DMA] * NB,
            [pltpu.SemaphoreType.DMA] * NB,
        ],
    )
    def k(x_hbm, srcr_hbm, dstr_hbm, ea_hbm, out0, out1,
          src_v, dst_v, ea_v, g_v, agg_sh, sem_idx, sem_in, sem_s):
        c = lax.axis_index("c")
        s = lax.axis_index("s")
        wid = s * NC + c
        base = wid * NCH

        def issue_idx(g, bi):
            pltpu.async_copy(srcr_hbm.at[wid, g], src_v.at[bi], sem_idx[bi])
            pltpu.async_copy(dstr_hbm.at[wid, g], dst_v.at[bi], sem_idx[bi])

        def wait_idx(bi):
            pltpu.make_async_copy(srcr_hbm.at[0, 0], src_v.at[bi],
                                  sem_idx[bi]).wait()
            pltpu.make_async_copy(dstr_hbm.at[0, 0], dst_v.at[bi],
                                  sem_idx[bi]).wait()

        def issue_loads(g, b, bi):
            pltpu.async_copy(ea_hbm.at[pl.ds((base + g) * C, C)], ea_v.at[b],
                             sem_in[b])
            pltpu.async_copy(x_hbm.at[src_v.at[bi, 0]], g_v.at[b], sem_in[b])

        def wait_loads(b):
            pltpu.make_async_copy(ea_hbm.at[pl.ds(0, C)], ea_v.at[b],
                                  sem_in[b]).wait()
            pltpu.make_async_copy(ea_hbm.at[pl.ds(0, C)], g_v.at[b],
                                  sem_in[b]).wait()

        def issue_scatter(b, bi):
            pltpu.async_copy(g_v.at[b], agg_sh.at[dst_v.at[bi, 0]], sem_s[b],
                             add=True)

        def wait_scatter(b):
            pltpu.make_async_copy(ea_hbm.at[pl.ds(0, C)], g_v.at[b],
                                  sem_s[b]).wait()

        zero = jnp.zeros((LANES,), jnp.float32)

        def zrow(r, carry):
            for j in range(DV):
                g_v[0, r, pl.ds(j * LANES, LANES)] = zero
            return carry

        lax.fori_loop(0, ZC, zrow, 0)
        for t in range(ZT):
            zc = s + t * NS

            @pl.when(jnp.logical_or(NZ % NS == 0, zc < NZ))
            def _():
                pltpu.sync_copy(g_v.at[0], agg_sh.at[pl.ds(zc * ZC, ZC)])

        plsc.subcore_barrier()

        for g0 in range(NBI - 1):
            issue_idx(g0, g0)
        for g0 in range(NB - 1):
            wait_idx(g0)
            issue_loads(g0, g0, g0)

        def step(t, carry):
            for u in range(NBI):
                g = t * NBI + u
                b = u % NB
                bi = u

                @pl.when(g < NCH)
                def _():
                    wait_loads(b)

                    def row(r, rc):
                        for j in range(DV):
                            sl = pl.ds(j * LANES, LANES)
                            g_v[b, r, sl] = jnp.maximum(
                                g_v[b, r, sl] + ea_v[b, r, sl], 0.0)
                        return rc

                    lax.fori_loop(0, C, row, 0)
                    issue_scatter(b, bi)

                    @pl.when(g + 2 < NCH)
                    def _():
                        @pl.when(g >= 1)
                        def _():
                            wait_scatter((b + NB - 1) % NB)

                        @pl.when(g + NBI - 1 < NCH)
                        def _():
                            issue_idx(g + NBI - 1, (u + NBI - 1) % NBI)

                        wait_idx((u + 2) % NBI)
                        issue_loads(g + 2, (b + 2) % NB, (u + 2) % NBI)

            return carry

        lax.fori_loop(0, T, step, 0)
        for b in range(NB):
            wait_scatter(b)
        plsc.subcore_barrier()

        for t in range(ZT):
            zc = s + t * NS

            @pl.when(jnp.logical_or(NZ % NS == 0, zc < NZ))
            def _():
                row0 = zc * ZC
                pltpu.sync_copy(agg_sh.at[pl.ds(row0, ZC)], g_v.at[0])

                @pl.when(c == 0)
                def _():
                    pltpu.sync_copy(g_v.at[0], out0.at[pl.ds(row0, ZC)])

                @pl.when(c == 1)
                def _():
                    pltpu.sync_copy(g_v.at[0], out1.at[pl.ds(row0, ZC)])

    return k


def kernel(x, edge_index, edge_attr, params):
    N, D = x.shape
    E = edge_index.shape[1]
    C = 40
    NCH = E // (NW * C)

    src_r = edge_index[0].reshape(NW, NCH, 1, C)
    dst_r = edge_index[1].reshape(NW, NCH, 1, C)

    h = x
    nl = len(params)
    for l, p in enumerate(params):
        ea = _ea_proj(edge_attr, p['eW'], p['eb'])
        sc = _sc_gather_scatter_fn(N, D, C, NCH)
        a0, a1 = sc(h, src_r, dst_r, ea)
        h = _mlp(h, a0, a1, p, final_relu=(l < nl - 1))
    return h


# speedup vs baseline: 4.6427x; 1.0122x over previous
import functools

import jax
import jax.numpy as jnp
from jax import lax
from jax.experimental import pallas as pl
from jax.experimental.pallas import tpu as pltpu
from jax.experimental.pallas import tpu_sc as plsc

NC = 2
NS = 16
NW = NC * NS
LANES = 16


def _ea_proj_body(attr_ref, w_ref, b_ref, out_ref):
    out_ref[...] = (
        jnp.dot(attr_ref[...], w_ref[...], preferred_element_type=jnp.float32)
        + b_ref[...]
    )


def _ea_proj(edge_attr, w, b, blk=2000):
    E, ED = edge_attr.shape
    D = w.shape[1]
    return pl.pallas_call(
        _ea_proj_body,
        grid=(E // blk,),
        in_specs=[
            pl.BlockSpec((blk, ED), lambda i: (i, 0)),
            pl.BlockSpec((ED, D), lambda i: (0, 0)),
            pl.BlockSpec((1, D), lambda i: (0, 0)),
        ],
        out_specs=pl.BlockSpec((blk, D), lambda i: (i, 0)),
        out_shape=jax.ShapeDtypeStruct((E, D), jnp.float32),
    )(edge_attr, w, b.reshape(1, D))


def _mlp_body(x_ref, a0_ref, a1_ref, w0, b0, w1, b1, w2, b2, out_ref, *,
              final_relu):
    h = x_ref[...] + a0_ref[...] + a1_ref[...]
    h = jnp.maximum(
        jnp.dot(h, w0[...], preferred_element_type=jnp.float32) + b0[...], 0.0)
    h = jnp.maximum(
        jnp.dot(h, w1[...], preferred_element_type=jnp.float32) + b1[...], 0.0)
    h = jnp.dot(h, w2[...], preferred_element_type=jnp.float32) + b2[...]
    if final_relu:
        h = jnp.maximum(h, 0.0)
    out_ref[...] = h


def _mlp(x, a0, a1, p, final_relu, blk=1000):
    N, D = x.shape
    mat = pl.BlockSpec((D, D), lambda i: (0, 0))
    vec = pl.BlockSpec((1, D), lambda i: (0, 0))
    rows = pl.BlockSpec((blk, D), lambda i: (i, 0))
    return pl.pallas_call(
        functools.partial(_mlp_body, final_relu=final_relu),
        grid=(N // blk,),
        in_specs=[rows, rows, rows, mat, vec, mat, vec, mat, vec],
        out_specs=rows,
        out_shape=jax.ShapeDtypeStruct((N, D), jnp.float32),
    )(x, a0, a1,
      p['W0'], p['b0'].reshape(1, D),
      p['W1'], p['b1'].reshape(1, D),
      p['W2'], p['b2'].reshape(1, D))


@functools.lru_cache(maxsize=None)
def _sc_gather_scatter_fn(N, D, C, NCH):
    NB = 3
    NBI = 6
    ZC = C
    NZ = N // ZC
    ZT = (NZ + NS - 1) // NS
    DV = D // LANES
    T = (NCH + NBI - 1) // NBI

    mesh = plsc.VectorSubcoreMesh(core_axis_name="c", subcore_axis_name="s")

    @functools.partial(
        pl.kernel,
        out_type=[jax.ShapeDtypeStruct((N, D), jnp.float32),
                  jax.ShapeDtypeStruct((N, D), jnp.float32)],
        mesh=mesh,
        scratch_types=[
            pltpu.VMEM((NBI, 1, C), jnp.int32),
            pltpu.VMEM((NBI, 1, C), jnp.int32),
            pltpu.VMEM((NB, C, D), jnp.float32),
            pltpu.VMEM((NB, C, D), jnp.float32),
            pltpu.VMEM_SHARED((N, D), jnp.float32),
            [pltpu.SemaphoreType.DMA] * NBI,
            [pltpu.SemaphoreType.DMA] * NB,
            [pltpu.SemaphoreType.DMA] * NB,
        ],
    )
    def k(x_hbm, srcr_hbm, dstr_hbm, ea_hbm, out0, out1,
          src_v, dst_v, ea_v, g_v, agg_sh, sem_idx, sem_in, sem_s):
        c = lax.axis_index("c")
        s = lax.axis_index("s")
        wid = s * NC + c
        base = wid * NCH

        def issue_idx(g, bi):
            pltpu.async_copy(srcr_hbm.at[wid, g], src_v.at[bi], sem_idx[bi])
            pltpu.async_copy(dstr_hbm.at[wid, g], dst_v.at[bi], sem_idx[bi])

        def wait_idx(bi):
            pltpu.make_async_copy(srcr_hbm.at[0, 0], src_v.at[bi],
                                  sem_idx[bi]).wait()
            pltpu.make_async_copy(dstr_hbm.at[0, 0], dst_v.at[bi],
                                  sem_idx[bi]).wait()

        def issue_loads(g, b, bi):
            pltpu.async_copy(ea_hbm.at[pl.ds((base + g) * C, C)], ea_v.at[b],
                             sem_in[b])
            pltpu.async_copy(x_hbm.at[src_v.at[bi, 0]], g_v.at[b], sem_in[b])

        def wait_loads(b):
            pltpu.make_async_copy(ea_hbm.at[pl.ds(0, C)], ea_v.at[b],
                                  sem_in[b]).wait()
            pltpu.make_async_copy(ea_hbm.at[pl.ds(0, C)], g_v.at[b],
                                  sem_in[b]).wait()

        def issue_scatter(b, bi):
            pltpu.async_copy(g_v.at[b], agg_sh.at[dst_v.at[bi, 0]], sem_s[b],
                             add=True)

        def wait_scatter(b):
            pltpu.make_async_copy(ea_hbm.at[pl.ds(0, C)], g_v.at[b],
                                  sem_s[b]).wait()

        zero = jnp.zeros((LANES,), jnp.float32)

        def zrow(r, carry):
            for j in range(DV):
                g_v[0, r, pl.ds(j * LANES, LANES)] = zero
            return carry

        lax.fori_loop(0, ZC, zrow, 0)
        for t in range(ZT):
            zc = s + t * NS

            @pl.when(jnp.logical_or(NZ % NS == 0, zc < NZ))
            def _():
                pltpu.async_copy(g_v.at[0], agg_sh.at[pl.ds(zc * ZC, ZC)],
                                 sem_s[0])
        for t in range(ZT):
            zc = s + t * NS

            @pl.when(jnp.logical_or(NZ % NS == 0, zc < NZ))
            def _():
                pltpu.make_async_copy(ea_hbm.at[pl.ds(0, ZC)], g_v.at[0],
                                      sem_s[0]).wait()
        plsc.subcore_barrier()

        for g0 in range(NBI - 1):
            issue_idx(g0, g0)
        for g0 in range(NB - 1):
            wait_idx(g0)
            issue_loads(g0, g0, g0)

        def step(t, carry):
            for u in range(NBI):
                g = t * NBI + u
                b = u % NB
                bi = u

                @pl.when(g < NCH)
                def _():
                    wait_loads(b)

                    def row(r, rc):
                        for j in range(DV):
                            sl = pl.ds(j * LANES, LANES)
                            g_v[b, r, sl] = jnp.maximum(
                                g_v[b, r, sl] + ea_v[b, r, sl], 0.0)
                        return rc

                    lax.fori_loop(0, C, row, 0)
                    issue_scatter(b, bi)

                    @pl.when(g + 2 < NCH)
                    def _():
                        @pl.when(g >= 1)
                        def _():
                            wait_scatter((b + NB - 1) % NB)

                        @pl.when(g + NBI - 1 < NCH)
                        def _():
                            issue_idx(g + NBI - 1, (u + NBI - 1) % NBI)

                        wait_idx((u + 2) % NBI)
                        issue_loads(g + 2, (b + 2) % NB, (u + 2) % NBI)

            return carry

        lax.fori_loop(0, T, step, 0)
        for b in range(NB):
            wait_scatter(b)
        plsc.subcore_barrier()

        def co_read(t, p):
            zc = s + t * NS

            @pl.when(jnp.logical_or(NZ % NS == 0, zc < NZ))
            def _():
                pltpu.async_copy(agg_sh.at[pl.ds(zc * ZC, ZC)], g_v.at[p],
                                 sem_in[p])

        def co_wait_read(t, p):
            zc = s + t * NS

            @pl.when(jnp.logical_or(NZ % NS == 0, zc < NZ))
            def _():
                pltpu.make_async_copy(ea_hbm.at[pl.ds(0, ZC)], g_v.at[p],
                                      sem_in[p]).wait()

        def co_write(t, p):
            zc = s + t * NS

            @pl.when(jnp.logical_or(NZ % NS == 0, zc < NZ))
            def _():
                row0 = zc * ZC

                @pl.when(c == 0)
                def _():
                    pltpu.async_copy(g_v.at[p], out0.at[pl.ds(row0, ZC)],
                                     sem_s[p])

                @pl.when(c == 1)
                def _():
                    pltpu.async_copy(g_v.at[p], out1.at[pl.ds(row0, ZC)],
                                     sem_s[p])

        def co_wait_write(t, p):
            zc = s + t * NS

            @pl.when(jnp.logical_or(NZ % NS == 0, zc < NZ))
            def _():
                pltpu.make_async_copy(ea_hbm.at[pl.ds(0, ZC)], g_v.at[p],
                                      sem_s[p]).wait()

        co_read(0, 0)
        for t in range(ZT):
            p = t % 2
            if t + 1 < ZT:
                if t >= 1:
                    co_wait_write(t - 1, (t + 1) % 2)
                co_read(t + 1, (t + 1) % 2)
            co_wait_read(t, p)
            co_write(t, p)
        for t in (ZT - 2, ZT - 1):
            if t >= 0:
                co_wait_write(t, t % 2)

    return k


def kernel(x, edge_index, edge_attr, params):
    N, D = x.shape
    E = edge_index.shape[1]
    C = 40
    NCH = E // (NW * C)

    src_r = edge_index[0].reshape(NW, NCH, 1, C)
    dst_r = edge_index[1].reshape(NW, NCH, 1, C)

    h = x
    nl = len(params)
    eas = [_ea_proj(edge_attr, p['eW'], p['eb']) for p in params]
    for l, p in enumerate(params):
        sc = _sc_gather_scatter_fn(N, D, C, NCH)
        a0, a1 = sc(h, src_r, dst_r, eas[l])
        h = _mlp(h, a0, a1, p, final_relu=(l < nl - 1))
    return h
